# Initial kernel scaffold; baseline (speedup 1.0000x reference)
import jax
import jax.numpy as jnp
from jax.experimental import pallas as pl


def kernel(points, dropout_weights):
    raise NotImplementedError("write your pallas kernel here")


# speedup vs baseline: 3.3963x; 3.3963x over previous
import functools

import jax
import jax.numpy as jnp
from jax import lax
from jax.experimental import pallas as pl
from jax.experimental.pallas import tpu as pltpu
from jax.experimental.pallas import tpu_sc as plsc

N = 16384
K = 8192
B = 16
F = 128
IBLK = 1024
JCHUNK = 512
WROW_R = 8
NTILES = 32
RPT = N // 4
CH = 128
NCHUNK = RPT // CH


def _rank_body(wcol_ref, wrow_ref, out_ref):
    i0 = pl.program_id(0) * IBLK
    wc = wcol_ref[...]
    ii = i0 + lax.broadcasted_iota(jnp.int32, (IBLK, 1), 0)
    acc = jnp.zeros((IBLK, 1), jnp.int32)
    for t in range(N // JCHUNK):
        r, c0 = divmod(t * JCHUNK, N // WROW_R)
        jv = wrow_ref[r:r + 1, c0:c0 + JCHUNK]
        jj = t * JCHUNK + lax.broadcasted_iota(jnp.int32, (1, JCHUNK), 1)
        gt = jv > wc
        ge = jv >= wc
        gt32 = jnp.where(gt, jnp.int32(1), jnp.int32(0))
        ge32 = jnp.where(ge, jnp.int32(1), jnp.int32(0))
        p32 = jnp.where(jj < ii, ge32, gt32)
        acc = acc + jnp.sum(p32, axis=1, keepdims=True)
    out_ref[...] = acc


def _ranks_tc(s):
    return pl.pallas_call(
        _rank_body,
        grid=(N // IBLK,),
        in_specs=[
            pl.BlockSpec((IBLK, 1), lambda i: (i, 0)),
            pl.BlockSpec((WROW_R, N // WROW_R), lambda i: (0, 0)),
        ],
        out_specs=pl.BlockSpec((IBLK, 1), lambda i: (i, 0)),
        out_shape=jax.ShapeDtypeStruct((N, 1), jnp.int32),
    )(s.reshape(N, 1), s.reshape(WROW_R, N // WROW_R))


def _sc_body(ranks_hbm, pts_hbm, out_hbm, ranks_v, idx_v, buf0, buf1, sem0, sem1):
    cid = lax.axis_index("c")
    sid = lax.axis_index("s")
    wid = sid * 2 + cid
    boff = sid * N
    lo = cid * RPT
    pltpu.sync_copy(ranks_hbm, ranks_v)
    iota16 = lax.iota(jnp.int32, 16)

    def build(k, carry):
        r = ranks_v[pl.ds(k * 16, 16)]
        rl = r - lo
        m = (rl >= 0) & (rl < RPT)
        rlc = rl & (RPT - 1)
        plsc.store_scatter(idx_v, [rlc], iota16 + (k * 16 + boff), mask=m)
        return carry

    lax.fori_loop(0, N // 16, build, 0)

    row0 = wid * RPT
    prev = None
    for g in range(NCHUNK):
        buf, sem = (buf0, sem0) if (g & 1) == 0 else (buf1, sem1)
        cp = pltpu.async_copy(pts_hbm.at[idx_v.at[pl.ds(g * CH, CH)]], buf, sem)
        if prev is not None:
            pcp, pbuf, pg = prev
            pcp.wait()
            pltpu.sync_copy(pbuf, out_hbm.at[pl.ds(row0 + pg * CH, CH)])
        prev = (cp, buf, g)
    pcp, pbuf, pg = prev
    pcp.wait()
    pltpu.sync_copy(pbuf, out_hbm.at[pl.ds(row0 + pg * CH, CH)])


@functools.cache
def _sc_gather():
    return pl.kernel(
        _sc_body,
        out_type=jax.ShapeDtypeStruct((B * K, F), jnp.float32),
        mesh=plsc.VectorSubcoreMesh(core_axis_name="c", subcore_axis_name="s",
                                    num_cores=2, num_subcores=16),
        compiler_params=pltpu.CompilerParams(needs_layout_passes=False),
        scratch_types=[
            pltpu.VMEM((N,), jnp.int32),
            pltpu.VMEM((RPT,), jnp.int32),
            pltpu.VMEM((CH, F), jnp.float32),
            pltpu.VMEM((CH, F), jnp.float32),
            pltpu.SemaphoreType.DMA,
            pltpu.SemaphoreType.DMA,
        ],
    )


def kernel(points, dropout_weights):
    s = jax.nn.sigmoid(dropout_weights)
    ranks = _ranks_tc(s).reshape(N)
    out = _sc_gather()(ranks, points.reshape(B * N, F))
    return out.reshape(B, K, F)


# speedup vs baseline: 4.2597x; 1.2542x over previous
import functools

import jax
import jax.numpy as jnp
from jax import lax
from jax.experimental import pallas as pl
from jax.experimental.pallas import tpu as pltpu
from jax.experimental.pallas import tpu_sc as plsc

N = 16384
K = 8192
B = 16
F = 128
IBLK = 1024
JCHUNK = 1024
WROW_R = 16
NTILES = 32
RPT = N // 4
CH = 128
NCHUNK = RPT // CH


def _rank_body(wcol_ref, wrow_ref, out_ref):
    ib = pl.program_id(0)
    i0 = ib * IBLK
    wc = wcol_ref[...]
    ii = i0 + lax.broadcasted_iota(jnp.int32, (IBLK, 1), 0)
    acc = jnp.zeros((IBLK, 1), jnp.float32)
    one = jnp.float32(1.0)
    zero = jnp.float32(0.0)
    for t in range(N // JCHUNK):
        jv = wrow_ref[t:t + 1, :]

        def below(jv=jv):
            return jnp.sum(jnp.where(jv >= wc, one, zero), axis=1, keepdims=True)

        def above(jv=jv):
            return jnp.sum(jnp.where(jv > wc, one, zero), axis=1, keepdims=True)

        def diag(jv=jv, t=t):
            jj = t * JCHUNK + lax.broadcasted_iota(jnp.int32, (1, JCHUNK), 1)
            gef = jnp.where(jv >= wc, one, zero)
            gtf = jnp.where(jv > wc, one, zero)
            p = jnp.where(jj < ii, gef, gtf)
            return jnp.sum(p, axis=1, keepdims=True)

        acc = acc + lax.cond(t == ib, diag,
                             lambda: lax.cond(t < ib, below, above))
    out_ref[...] = acc.astype(jnp.int32)


def _ranks_tc(s):
    return pl.pallas_call(
        _rank_body,
        grid=(N // IBLK,),
        in_specs=[
            pl.BlockSpec((IBLK, 1), lambda i: (i, 0)),
            pl.BlockSpec((WROW_R, N // WROW_R), lambda i: (0, 0)),
        ],
        out_specs=pl.BlockSpec((IBLK, 1), lambda i: (i, 0)),
        out_shape=jax.ShapeDtypeStruct((N, 1), jnp.int32),
    )(s.reshape(N, 1), s.reshape(WROW_R, N // WROW_R))


def _sc_body(ranks_hbm, pts_hbm, out_hbm, ranks_v, idx_v, buf0, buf1, sem0, sem1):
    cid = lax.axis_index("c")
    sid = lax.axis_index("s")
    wid = sid * 2 + cid
    boff = sid * N
    lo = cid * RPT
    pltpu.sync_copy(ranks_hbm, ranks_v)
    iota16 = lax.iota(jnp.int32, 16)

    def build(k, carry):
        r = ranks_v[pl.ds(k * 16, 16)]
        rl = r - lo
        m = (rl >= 0) & (rl < RPT)
        rlc = rl & (RPT - 1)
        plsc.store_scatter(idx_v, [rlc], iota16 + (k * 16 + boff), mask=m)
        return carry

    lax.fori_loop(0, N // 16, build, 0)

    row0 = wid * RPT
    prev = None
    for g in range(NCHUNK):
        buf, sem = (buf0, sem0) if (g & 1) == 0 else (buf1, sem1)
        cp = pltpu.async_copy(pts_hbm.at[idx_v.at[pl.ds(g * CH, CH)]], buf, sem)
        if prev is not None:
            pcp, pbuf, pg = prev
            pcp.wait()
            pltpu.sync_copy(pbuf, out_hbm.at[pl.ds(row0 + pg * CH, CH)])
        prev = (cp, buf, g)
    pcp, pbuf, pg = prev
    pcp.wait()
    pltpu.sync_copy(pbuf, out_hbm.at[pl.ds(row0 + pg * CH, CH)])


@functools.cache
def _sc_gather():
    return pl.kernel(
        _sc_body,
        out_type=jax.ShapeDtypeStruct((B * K, F), jnp.float32),
        mesh=plsc.VectorSubcoreMesh(core_axis_name="c", subcore_axis_name="s",
                                    num_cores=2, num_subcores=16),
        compiler_params=pltpu.CompilerParams(needs_layout_passes=False),
        scratch_types=[
            pltpu.VMEM((N,), jnp.int32),
            pltpu.VMEM((RPT,), jnp.int32),
            pltpu.VMEM((CH, F), jnp.float32),
            pltpu.VMEM((CH, F), jnp.float32),
            pltpu.SemaphoreType.DMA,
            pltpu.SemaphoreType.DMA,
        ],
    )


def kernel(points, dropout_weights):
    s = jax.nn.sigmoid(dropout_weights)
    ranks = _ranks_tc(s).reshape(N)
    out = _sc_gather()(ranks, points.reshape(B * N, F))
    return out.reshape(B, K, F)


# speedup vs baseline: 5.9193x; 1.3896x over previous
import functools

import jax
import jax.numpy as jnp
from jax import lax
from jax.experimental import pallas as pl
from jax.experimental.pallas import tpu as pltpu
from jax.experimental.pallas import tpu_sc as plsc

N = 16384
K = 8192
B = 16
F = 128
IBLK = 1024
JCHUNK = 1024
WROW_R = 16
NTILES = 32
RPT = N // 4
CH = 128
NCHUNK = RPT // CH


def _rank_body(wcol_ref, wrow_ref, out_ref):
    ib = pl.program_id(0)
    wc = wcol_ref[...]
    wcu = lax.bitcast_convert_type(
        lax.bitcast_convert_type(wc, jnp.int32) + 1, jnp.float32)
    ii = ib * IBLK + lax.broadcasted_iota(jnp.int32, (IBLK, 1), 0)
    acc = jnp.zeros((IBLK, 1), jnp.float32)
    one = jnp.float32(1.0)
    zero = jnp.float32(0.0)
    for t in range(N // JCHUNK):
        jv = wrow_ref[t:t + 1, :]
        thr = jnp.where(t < ib, wc, wcu)
        p = jnp.where(jv >= thr, one, zero)
        acc = acc + jnp.sum(p, axis=1, keepdims=True)
    jd = wrow_ref[pl.ds(ib, 1), :]
    jjd = ib * JCHUNK + lax.broadcasted_iota(jnp.int32, (1, JCHUNK), 1)
    tie = jnp.where((jd == wc) & (jjd < ii), one, zero)
    acc = acc + jnp.sum(tie, axis=1, keepdims=True)
    out_ref[...] = acc.astype(jnp.int32)


def _ranks_tc(s):
    return pl.pallas_call(
        _rank_body,
        grid=(N // IBLK,),
        in_specs=[
            pl.BlockSpec((IBLK, 1), lambda i: (i, 0)),
            pl.BlockSpec((WROW_R, N // WROW_R), lambda i: (0, 0)),
        ],
        out_specs=pl.BlockSpec((IBLK, 1), lambda i: (i, 0)),
        out_shape=jax.ShapeDtypeStruct((N, 1), jnp.int32),
    )(s.reshape(N, 1), s.reshape(WROW_R, N // WROW_R))


def _sc_body(ranks_hbm, pts_hbm, out_hbm, ranks_v, idx_v, buf0, buf1, sem0, sem1):
    cid = lax.axis_index("c")
    sid = lax.axis_index("s")
    wid = sid * 2 + cid
    boff = sid * N
    lo = cid * RPT
    pltpu.sync_copy(ranks_hbm, ranks_v)
    iota16 = lax.iota(jnp.int32, 16)

    def build(k, carry):
        r = ranks_v[pl.ds(k * 16, 16)]
        rl = r - lo
        m = (rl >= 0) & (rl < RPT)
        rlc = rl & (RPT - 1)
        plsc.store_scatter(idx_v, [rlc], iota16 + (k * 16 + boff), mask=m)
        return carry

    lax.fori_loop(0, N // 16, build, 0)

    row0 = wid * RPT
    prev = None
    for g in range(NCHUNK):
        buf, sem = (buf0, sem0) if (g & 1) == 0 else (buf1, sem1)
        cp = pltpu.async_copy(pts_hbm.at[idx_v.at[pl.ds(g * CH, CH)]], buf, sem)
        if prev is not None:
            pcp, pbuf, pg = prev
            pcp.wait()
            pltpu.sync_copy(pbuf, out_hbm.at[pl.ds(row0 + pg * CH, CH)])
        prev = (cp, buf, g)
    pcp, pbuf, pg = prev
    pcp.wait()
    pltpu.sync_copy(pbuf, out_hbm.at[pl.ds(row0 + pg * CH, CH)])


@functools.cache
def _sc_gather():
    return pl.kernel(
        _sc_body,
        out_type=jax.ShapeDtypeStruct((B * K, F), jnp.float32),
        mesh=plsc.VectorSubcoreMesh(core_axis_name="c", subcore_axis_name="s",
                                    num_cores=2, num_subcores=16),
        compiler_params=pltpu.CompilerParams(needs_layout_passes=False),
        scratch_types=[
            pltpu.VMEM((N,), jnp.int32),
            pltpu.VMEM((RPT,), jnp.int32),
            pltpu.VMEM((CH, F), jnp.float32),
            pltpu.VMEM((CH, F), jnp.float32),
            pltpu.SemaphoreType.DMA,
            pltpu.SemaphoreType.DMA,
        ],
    )


def kernel(points, dropout_weights):
    s = jax.nn.sigmoid(dropout_weights)
    ranks = _ranks_tc(s).reshape(N)
    out = _sc_gather()(ranks, points.reshape(B * N, F))
    return out.reshape(B, K, F)


# speedup vs baseline: 6.6744x; 1.1276x over previous
import functools

import jax
import jax.numpy as jnp
from jax import lax
from jax.experimental import pallas as pl
from jax.experimental.pallas import tpu as pltpu
from jax.experimental.pallas import tpu_sc as plsc

N = 16384
K = 8192
B = 16
F = 128
IBLK = 1024
JCHUNK = 1024
WROW_R = 16
NTILES = 32
RPT = N // 4
CH = 128
NCHUNK = RPT // CH


def _rank_body(wcol_ref, wrow_ref, out_ref):
    one = jnp.float32(1.0)
    zero = jnp.float32(0.0)
    for ibk in range(N // IBLK):
        wc = wcol_ref[ibk * IBLK:(ibk + 1) * IBLK, :]
        wcu = lax.bitcast_convert_type(
            lax.bitcast_convert_type(wc, jnp.int32) + 1, jnp.float32)
        acc = jnp.zeros((IBLK, 1), jnp.float32)
        for t in range(N // JCHUNK):
            jv = wrow_ref[t:t + 1, :]
            if t < ibk:
                p = jnp.where(jv >= wc, one, zero)
            elif t > ibk:
                p = jnp.where(jv >= wcu, one, zero)
            else:
                ii = ibk * IBLK + lax.broadcasted_iota(jnp.int32, (IBLK, 1), 0)
                jj = t * JCHUNK + lax.broadcasted_iota(jnp.int32, (1, JCHUNK), 1)
                gef = jnp.where(jv >= wc, one, zero)
                gtf = jnp.where(jv >= wcu, one, zero)
                p = jnp.where(jj < ii, gef, gtf)
            acc = acc + jnp.sum(p, axis=1, keepdims=True)
        out_ref[ibk * IBLK:(ibk + 1) * IBLK, :] = acc.astype(jnp.int32)


def _ranks_tc(s):
    return pl.pallas_call(
        _rank_body,
        grid=(1,),
        in_specs=[
            pl.BlockSpec((N, 1), lambda i: (0, 0)),
            pl.BlockSpec((WROW_R, N // WROW_R), lambda i: (0, 0)),
        ],
        out_specs=pl.BlockSpec((N, 1), lambda i: (0, 0)),
        out_shape=jax.ShapeDtypeStruct((N, 1), jnp.int32),
    )(s.reshape(N, 1), s.reshape(WROW_R, N // WROW_R))


def _sc_body(ranks_hbm, pts_hbm, out_hbm, ranks_v, idx_v, buf0, buf1, sem0, sem1):
    cid = lax.axis_index("c")
    sid = lax.axis_index("s")
    wid = sid * 2 + cid
    boff = sid * N
    lo = cid * RPT
    pltpu.sync_copy(ranks_hbm, ranks_v)
    iota16 = lax.iota(jnp.int32, 16)

    def build(k, carry):
        r = ranks_v[pl.ds(k * 16, 16)]
        rl = r - lo
        m = (rl >= 0) & (rl < RPT)
        rlc = rl & (RPT - 1)
        plsc.store_scatter(idx_v, [rlc], iota16 + (k * 16 + boff), mask=m)
        return carry

    lax.fori_loop(0, N // 16, build, 0)

    row0 = wid * RPT
    prev = None
    for g in range(NCHUNK):
        buf, sem = (buf0, sem0) if (g & 1) == 0 else (buf1, sem1)
        cp = pltpu.async_copy(pts_hbm.at[idx_v.at[pl.ds(g * CH, CH)]], buf, sem)
        if prev is not None:
            pcp, pbuf, pg = prev
            pcp.wait()
            pltpu.sync_copy(pbuf, out_hbm.at[pl.ds(row0 + pg * CH, CH)])
        prev = (cp, buf, g)
    pcp, pbuf, pg = prev
    pcp.wait()
    pltpu.sync_copy(pbuf, out_hbm.at[pl.ds(row0 + pg * CH, CH)])


@functools.cache
def _sc_gather():
    return pl.kernel(
        _sc_body,
        out_type=jax.ShapeDtypeStruct((B * K, F), jnp.float32),
        mesh=plsc.VectorSubcoreMesh(core_axis_name="c", subcore_axis_name="s",
                                    num_cores=2, num_subcores=16),
        compiler_params=pltpu.CompilerParams(needs_layout_passes=False),
        scratch_types=[
            pltpu.VMEM((N,), jnp.int32),
            pltpu.VMEM((RPT,), jnp.int32),
            pltpu.VMEM((CH, F), jnp.float32),
            pltpu.VMEM((CH, F), jnp.float32),
            pltpu.SemaphoreType.DMA,
            pltpu.SemaphoreType.DMA,
        ],
    )


def kernel(points, dropout_weights):
    s = jax.nn.sigmoid(dropout_weights)
    ranks = _ranks_tc(s).reshape(N)
    out = _sc_gather()(ranks, points.reshape(B * N, F))
    return out.reshape(B, K, F)


# speedup vs baseline: 7.8889x; 1.1820x over previous
import functools

import jax
import jax.numpy as jnp
from jax import lax
from jax.experimental import pallas as pl
from jax.experimental.pallas import tpu as pltpu
from jax.experimental.pallas import tpu_sc as plsc

N = 16384
K = 8192
B = 16
F = 128
IBLK = 1024
JCHUNK = 1024
WROW_R = 16
NTILES = 32
RPT = N // 4
CH = 128
NCHUNK = RPT // CH


def _rank_body(wcol_ref, wrow_ref, out_ref):
    one = jnp.float32(1.0)
    zero = jnp.float32(0.0)
    nb = N // IBLK
    acc_c = [jnp.zeros((1, JCHUNK), jnp.float32) for _ in range(nb)]
    for ibk in range(nb):
        wc = wcol_ref[ibk * IBLK:(ibk + 1) * IBLK, :]
        wcu = lax.bitcast_convert_type(
            lax.bitcast_convert_type(wc, jnp.int32) + 1, jnp.float32)
        jv = wrow_ref[ibk:ibk + 1, :]
        ii = ibk * IBLK + lax.broadcasted_iota(jnp.int32, (IBLK, 1), 0)
        jj = ibk * JCHUNK + lax.broadcasted_iota(jnp.int32, (1, JCHUNK), 1)
        gef = jnp.where(jv >= wc, one, zero)
        gtf = jnp.where(jv >= wcu, one, zero)
        p = jnp.where(jj < ii, gef, gtf)
        acc = jnp.sum(p, axis=1, keepdims=True)
        for t in range(ibk + 1, nb):
            jv = wrow_ref[t:t + 1, :]
            p = jnp.where(jv >= wcu, one, zero)
            acc = acc + jnp.sum(p, axis=1, keepdims=True)
            acc_c[t] = acc_c[t] + jnp.sum(p, axis=0, keepdims=True)
        low = jnp.float32(ibk * IBLK) - acc_c[ibk]
        rank = acc + jnp.reshape(low, (IBLK, 1))
        out_ref[ibk * IBLK:(ibk + 1) * IBLK, :] = rank.astype(jnp.int32)


def _ranks_tc(s):
    return pl.pallas_call(
        _rank_body,
        grid=(1,),
        in_specs=[
            pl.BlockSpec((N, 1), lambda i: (0, 0)),
            pl.BlockSpec((WROW_R, N // WROW_R), lambda i: (0, 0)),
        ],
        out_specs=pl.BlockSpec((N, 1), lambda i: (0, 0)),
        out_shape=jax.ShapeDtypeStruct((N, 1), jnp.int32),
    )(s.reshape(N, 1), s.reshape(WROW_R, N // WROW_R))


def _sc_body(ranks_hbm, pts_hbm, out_hbm, ranks_v, idx_v, buf0, buf1, sem0, sem1):
    cid = lax.axis_index("c")
    sid = lax.axis_index("s")
    wid = sid * 2 + cid
    boff = sid * N
    lo = cid * RPT
    pltpu.sync_copy(ranks_hbm, ranks_v)
    iota16 = lax.iota(jnp.int32, 16)

    def build(k, carry):
        r = ranks_v[pl.ds(k * 16, 16)]
        rl = r - lo
        m = (rl >= 0) & (rl < RPT)
        rlc = rl & (RPT - 1)
        plsc.store_scatter(idx_v, [rlc], iota16 + (k * 16 + boff), mask=m)
        return carry

    lax.fori_loop(0, N // 16, build, 0)

    row0 = wid * RPT
    prev = None
    for g in range(NCHUNK):
        buf, sem = (buf0, sem0) if (g & 1) == 0 else (buf1, sem1)
        cp = pltpu.async_copy(pts_hbm.at[idx_v.at[pl.ds(g * CH, CH)]], buf, sem)
        if prev is not None:
            pcp, pbuf, pg = prev
            pcp.wait()
            pltpu.sync_copy(pbuf, out_hbm.at[pl.ds(row0 + pg * CH, CH)])
        prev = (cp, buf, g)
    pcp, pbuf, pg = prev
    pcp.wait()
    pltpu.sync_copy(pbuf, out_hbm.at[pl.ds(row0 + pg * CH, CH)])


@functools.cache
def _sc_gather():
    return pl.kernel(
        _sc_body,
        out_type=jax.ShapeDtypeStruct((B * K, F), jnp.float32),
        mesh=plsc.VectorSubcoreMesh(core_axis_name="c", subcore_axis_name="s",
                                    num_cores=2, num_subcores=16),
        compiler_params=pltpu.CompilerParams(needs_layout_passes=False),
        scratch_types=[
            pltpu.VMEM((N,), jnp.int32),
            pltpu.VMEM((RPT,), jnp.int32),
            pltpu.VMEM((CH, F), jnp.float32),
            pltpu.VMEM((CH, F), jnp.float32),
            pltpu.SemaphoreType.DMA,
            pltpu.SemaphoreType.DMA,
        ],
    )


def kernel(points, dropout_weights):
    s = jax.nn.sigmoid(dropout_weights)
    ranks = _ranks_tc(s).reshape(N)
    out = _sc_gather()(ranks, points.reshape(B * N, F))
    return out.reshape(B, K, F)
